# baseline (speedup 1.0000x reference)
import jax
import jax.numpy as jnp
from jax.experimental import pallas as pl
from jax.experimental.pallas import tpu as pltpu

_RMAX = 4096
_NBUF = 8
_K = 4


def _schedule(B, S):
    sizes = [512] * 4 + [1024] * 2
    assert sum(sizes) == B
    tail = S - B
    big, rem = divmod(tail - 4096, _RMAX)
    assert rem == 0
    sizes += [_RMAX] * big + [2048, 1024, 512, 512]
    assert sum(sizes) == S
    blocks = []
    off = 0
    for sz in sizes:
        blocks.append((off, sz))
        off += sz
    return blocks


def _copy_body(emb, lab, eq, lq, out_eq, out_lq, vb, vlab, fsem, dsem, lsem):
    S, D = out_eq.shape
    B = emb.shape[0]
    blocks = _schedule(B, S)
    nb = len(blocks)

    fills = [
        pltpu.make_async_copy(
            (emb if off + sz <= B else eq).at[pl.ds(off, sz)],
            vb.at[b % _NBUF, pl.ds(0, sz)],
            fsem.at[b % _NBUF],
        )
        for b, (off, sz) in enumerate(blocks)
    ]
    drains = [
        pltpu.make_async_copy(
            vb.at[b % _NBUF, pl.ds(0, sz)],
            out_eq.at[pl.ds(off, sz)],
            dsem.at[b % _NBUF],
        )
        for b, (off, sz) in enumerate(blocks)
    ]
    rl = lab.shape[0]
    ltail = lq.shape[0] - rl
    lfill1 = pltpu.make_async_copy(lab, vlab.at[pl.ds(0, rl)], lsem.at[0])
    lfill2 = pltpu.make_async_copy(
        lq.at[pl.ds(rl, ltail)], vlab.at[pl.ds(rl, ltail)], lsem.at[0]
    )
    ldrain = pltpu.make_async_copy(vlab, out_lq, lsem.at[1])

    lfill1.start()
    lfill2.start()
    for b in range(min(_NBUF, nb)):
        fills[b].start()
    lfill1.wait()
    lfill2.wait()
    ldrain.start()
    waited = -1
    for b in range(nb):
        fills[b].wait()
        drains[b].start()
        j = b - _K
        if j >= 0 and j + _NBUF < nb:
            drains[j].wait()
            fills[j + _NBUF].start()
            waited = j
    for b in range(waited + 1, nb):
        drains[b].wait()
    ldrain.wait()


def kernel(embeddings, labels, embed_queue, label_queue):
    B, D = embeddings.shape
    S = embed_queue.shape[0]
    lab2 = labels.reshape(B // 128, 128)
    lq2 = label_queue.reshape(S // 128, 128)
    out_eq, out_lq = pl.pallas_call(
        _copy_body,
        in_specs=[pl.BlockSpec(memory_space=pl.ANY)] * 4,
        out_specs=[pl.BlockSpec(memory_space=pl.ANY)] * 2,
        out_shape=[
            jax.ShapeDtypeStruct(embed_queue.shape, embed_queue.dtype),
            jax.ShapeDtypeStruct(lq2.shape, lq2.dtype),
        ],
        scratch_shapes=[
            pltpu.VMEM((_NBUF, _RMAX, D), embed_queue.dtype),
            pltpu.VMEM((S // 128, 128), label_queue.dtype),
            pltpu.SemaphoreType.DMA((_NBUF,)),
            pltpu.SemaphoreType.DMA((_NBUF,)),
            pltpu.SemaphoreType.DMA((2,)),
        ],
    )(embeddings, lab2, embed_queue, lq2)
    new_ptr = jnp.array([B % S], dtype=jnp.int32)
    return out_eq, out_lq.reshape(S), new_ptr


# speedup vs baseline: 1.0542x; 1.0542x over previous
import jax
import jax.numpy as jnp
from jax.experimental import pallas as pl
from jax.experimental.pallas import tpu as pltpu

_R = 4096
_NBUF = 8
_K = 4


def _copy_body(emb, lab, eq, lq, out_eq, out_lq, vb, vlab, fsem, dsem, lsem):
    S, D = out_eq.shape
    B = emb.shape[0]
    nb = S // _R
    nb_emb = B // _R

    fills = [
        pltpu.make_async_copy(
            (emb if b < nb_emb else eq).at[pl.ds(b * _R, _R)],
            vb.at[b % _NBUF],
            fsem.at[b % _NBUF],
        )
        for b in range(nb)
    ]
    drains = [
        pltpu.make_async_copy(
            vb.at[b % _NBUF],
            out_eq.at[pl.ds(b * _R, _R)],
            dsem.at[b % _NBUF],
        )
        for b in range(nb)
    ]
    rl = lab.shape[0]
    ltail = lq.shape[0] - rl
    lfill1 = pltpu.make_async_copy(lab, vlab.at[pl.ds(0, rl)], lsem.at[0])
    lfill2 = pltpu.make_async_copy(
        lq.at[pl.ds(rl, ltail)], vlab.at[pl.ds(rl, ltail)], lsem.at[0]
    )
    ldrain = pltpu.make_async_copy(vlab, out_lq, lsem.at[1])

    lfill1.start()
    lfill2.start()
    for b in range(_NBUF):
        fills[b].start()
    lfill1.wait()
    lfill2.wait()
    ldrain.start()
    for b in range(nb):
        fills[b].wait()
        drains[b].start()
        j = b - _K
        if j >= 0 and j + _NBUF < nb:
            drains[j].wait()
            fills[j + _NBUF].start()
    waited = [j for j in range(nb) if j + _NBUF < nb and j <= nb - 1 - _K]
    first_unwaited = (waited[-1] + 1) if waited else 0
    for b in range(first_unwaited, nb):
        drains[b].wait()
    ldrain.wait()


def kernel(embeddings, labels, embed_queue, label_queue):
    B, D = embeddings.shape
    S = embed_queue.shape[0]
    lab2 = labels.reshape(B // 128, 128)
    lq2 = label_queue.reshape(S // 128, 128)
    out_eq, out_lq = pl.pallas_call(
        _copy_body,
        in_specs=[pl.BlockSpec(memory_space=pl.ANY)] * 4,
        out_specs=[pl.BlockSpec(memory_space=pl.ANY)] * 2,
        out_shape=[
            jax.ShapeDtypeStruct(embed_queue.shape, embed_queue.dtype),
            jax.ShapeDtypeStruct(lq2.shape, lq2.dtype),
        ],
        scratch_shapes=[
            pltpu.VMEM((_NBUF, _R, D), embed_queue.dtype),
            pltpu.VMEM((S // 128, 128), label_queue.dtype),
            pltpu.SemaphoreType.DMA((_NBUF,)),
            pltpu.SemaphoreType.DMA((_NBUF,)),
            pltpu.SemaphoreType.DMA((2,)),
        ],
    )(embeddings, lab2, embed_queue, lq2)
    new_ptr = jnp.array([B % S], dtype=jnp.int32)
    return out_eq, out_lq.reshape(S), new_ptr


# speedup vs baseline: 1.0826x; 1.0270x over previous
import jax
import jax.numpy as jnp
from jax.experimental import pallas as pl
from jax.experimental.pallas import tpu as pltpu

_R = 4096
_NBUF = 12
_K = 6


def _copy_body(emb, lab, eq, lq, out_eq, out_lq, vb, vlab, fsem, dsem, lsem):
    S, D = out_eq.shape
    B = emb.shape[0]
    nb = S // _R
    nb_emb = B // _R

    fills = [
        pltpu.make_async_copy(
            (emb if b < nb_emb else eq).at[pl.ds(b * _R, _R)],
            vb.at[b % _NBUF],
            fsem.at[b % _NBUF],
        )
        for b in range(nb)
    ]
    drains = [
        pltpu.make_async_copy(
            vb.at[b % _NBUF],
            out_eq.at[pl.ds(b * _R, _R)],
            dsem.at[b % _NBUF],
        )
        for b in range(nb)
    ]
    rl = lab.shape[0]
    ltail = lq.shape[0] - rl
    lfill1 = pltpu.make_async_copy(lab, vlab.at[pl.ds(0, rl)], lsem.at[0])
    lfill2 = pltpu.make_async_copy(
        lq.at[pl.ds(rl, ltail)], vlab.at[pl.ds(rl, ltail)], lsem.at[0]
    )
    ldrain = pltpu.make_async_copy(vlab, out_lq, lsem.at[1])

    lfill1.start()
    lfill2.start()
    for b in range(_NBUF):
        fills[b].start()
    lfill1.wait()
    lfill2.wait()
    ldrain.start()
    for b in range(nb):
        fills[b].wait()
        drains[b].start()
        j = b - _K
        if j >= 0 and j + _NBUF < nb:
            drains[j].wait()
            fills[j + _NBUF].start()
    waited = [j for j in range(nb) if j + _NBUF < nb and j <= nb - 1 - _K]
    first_unwaited = (waited[-1] + 1) if waited else 0
    for b in range(first_unwaited, nb):
        drains[b].wait()
    ldrain.wait()


def kernel(embeddings, labels, embed_queue, label_queue):
    B, D = embeddings.shape
    S = embed_queue.shape[0]
    lab2 = labels.reshape(B // 128, 128)
    lq2 = label_queue.reshape(S // 128, 128)
    out_eq, out_lq = pl.pallas_call(
        _copy_body,
        in_specs=[pl.BlockSpec(memory_space=pl.ANY)] * 4,
        out_specs=[pl.BlockSpec(memory_space=pl.ANY)] * 2,
        out_shape=[
            jax.ShapeDtypeStruct(embed_queue.shape, embed_queue.dtype),
            jax.ShapeDtypeStruct(lq2.shape, lq2.dtype),
        ],
        scratch_shapes=[
            pltpu.VMEM((_NBUF, _R, D), embed_queue.dtype),
            pltpu.VMEM((S // 128, 128), label_queue.dtype),
            pltpu.SemaphoreType.DMA((_NBUF,)),
            pltpu.SemaphoreType.DMA((_NBUF,)),
            pltpu.SemaphoreType.DMA((2,)),
        ],
    )(embeddings, lab2, embed_queue, lq2)
    new_ptr = jnp.array([B % S], dtype=jnp.int32)
    return out_eq, out_lq.reshape(S), new_ptr
